# baseline (speedup 1.0000x reference)
import jax
import jax.numpy as jnp
from jax import lax
from jax.experimental import pallas as pl
from jax.experimental.pallas import tpu as pltpu
from jax.experimental.pallas import tpu_sc as plsc

N = 10000
D = 256
T = 160000
MARGIN = 1.0

NC = 2
NS = 16
NW = NC * NS
TPW = T // NW
CHUNK = 40
NCHUNK = TPW // CHUNK
NBUF = 4
PRE = NBUF - 1
NMAIN = (NCHUNK - PRE) // NBUF
NTAIL = NCHUNK - NBUF * NMAIN
NLANE = 16
NSEG = D // (2 * NLANE)


def _sc_kernel(h1, h2, h3, trt, out,
               idx_v,
               av0, pv0, nv0, av1, pv1, nv1, av2, pv2, nv2, av3, pv3, nv3,
               acc_v, sem0, sem1, sem2, sem3):
    cid = lax.axis_index("c")
    sid = lax.axis_index("s")
    wid = sid * NC + cid
    base = wid * TPW

    pltpu.sync_copy(trt.at[pl.ds(base, TPW)], idx_v.at[pl.ds(0, TPW)])
    pltpu.sync_copy(trt.at[pl.ds(T + base, TPW)], idx_v.at[pl.ds(TPW, TPW)])
    pltpu.sync_copy(trt.at[pl.ds(2 * T + base, TPW)], idx_v.at[pl.ds(2 * TPW, TPW)])

    rings = (
        ((av0, pv0, nv0), sem0),
        ((av1, pv1, nv1), sem1),
        ((av2, pv2, nv2), sem2),
        ((av3, pv3, nv3), sem3),
    )

    def copies(g, bufs, sem):
        off = g * CHUNK
        ba, bp, bn = bufs
        return (
            pltpu.make_async_copy(h1.at[idx_v.at[pl.ds(off, CHUNK)]], ba, sem),
            pltpu.make_async_copy(h2.at[idx_v.at[pl.ds(TPW + off, CHUNK)]], bp, sem),
            pltpu.make_async_copy(h3.at[idx_v.at[pl.ds(2 * TPW + off, CHUNK)]], bn, sem),
        )

    def start(g, bufs, sem):
        for cp in copies(g, bufs, sem):
            cp.start()

    def wait(g, bufs, sem):
        for cp in copies(g, bufs, sem):
            cp.wait()

    def compute(bufs, acc):
        ba, bp, bn = bufs

        mask_hi = jnp.full((NLANE,), -65536, jnp.int32)
        shift16 = jnp.full((NLANE,), 16, jnp.int32)

        def one_triplet(t):
            part = jnp.zeros((NLANE,), jnp.float32)
            for j in range(NSEG):
                sl = pl.ds(j * NLANE, NLANE)
                a = plsc.bitcast(ba[t, sl], jnp.bfloat16)
                p = plsc.bitcast(bp[t, sl], jnp.bfloat16)
                q = plsc.bitcast(bn[t, sl], jnp.bfloat16)
                d1 = a - p
                d2 = a - q
                s = d1 - d2
                w = d1 + d2
                si = plsc.bitcast(s, jnp.int32)
                s_lo = plsc.bitcast(lax.shift_left(si, shift16), jnp.float32)
                s_hi = plsc.bitcast(lax.bitwise_and(si, mask_hi), jnp.float32)
                w_lo, w_hi = plsc.unpack(w, format=plsc.PackFormat.INTERLEAVED)
                part = part + s_lo * w_lo
                part = part + s_hi * w_hi
            c = jnp.sum(part)
            return jnp.maximum(c + MARGIN, 0.0)

        def t_body(t, acc2):
            return acc2 + one_triplet(t)

        return lax.fori_loop(0, CHUNK, t_body, acc)

    for b in range(PRE):
        start(b, *rings[b])

    def ring_body(i, acc):
        g0 = NBUF * i
        for b in range(NBUF):
            g = g0 + b
            bufs, sem = rings[b]
            wait(g, bufs, sem)
            start(g + PRE, *rings[(b + PRE) % NBUF])
            acc = compute(bufs, acc)
        return acc

    acc = lax.fori_loop(0, NMAIN, ring_body, jnp.float32(0.0))
    for k in range(NTAIL):
        g = NBUF * NMAIN + k
        bufs, sem = rings[g % NBUF]
        wait(g, bufs, sem)
        if g + PRE < NCHUNK:
            start(g + PRE, *rings[(g + PRE) % NBUF])
        acc = compute(bufs, acc)
    total = acc

    acc_v[...] = jnp.full((NLANE,), total, jnp.float32)
    pltpu.sync_copy(acc_v, out.at[wid])


PACK_ROWS = 400


def _pack_body(x1, x2, x3, o1, o2, o3):
    for x, o in ((x1, o1), (x2, o2), (x3, o3)):
        u = lax.bitcast_convert_type(x[...].astype(jnp.bfloat16), jnp.uint16)
        lo = u[:, : D // 2].astype(jnp.uint32)
        hi = u[:, D // 2:].astype(jnp.uint32)
        o[...] = lax.bitcast_convert_type(lo | (hi << 16), jnp.int32)


def _pack_tables(h1, h2, h3):
    spec_in = pl.BlockSpec((PACK_ROWS, D), lambda i: (i, 0))
    spec_out = pl.BlockSpec((PACK_ROWS, D // 2), lambda i: (i, 0))
    out = jax.ShapeDtypeStruct((N, D // 2), jnp.int32)
    return pl.pallas_call(
        _pack_body,
        grid=(N // PACK_ROWS,),
        in_specs=[spec_in] * 3,
        out_specs=[spec_out] * 3,
        out_shape=[out] * 3,
    )(h1, h2, h3)


@jax.jit
def kernel(h_c1, h_c2, h_c3, triplets):
    trt = triplets.astype(jnp.int32).T.reshape(3 * T)

    mesh = plsc.VectorSubcoreMesh(core_axis_name="c", subcore_axis_name="s",
                                  num_cores=NC, num_subcores=NS)
    run = pl.kernel(
        _sc_kernel,
        out_type=jax.ShapeDtypeStruct((NW, NLANE), jnp.float32),
        mesh=mesh,
        compiler_params=pltpu.CompilerParams(needs_layout_passes=False),
        scratch_types=[
            pltpu.VMEM((3 * TPW,), jnp.int32),
            pltpu.VMEM((CHUNK, D // 2), jnp.int32),
            pltpu.VMEM((CHUNK, D // 2), jnp.int32),
            pltpu.VMEM((CHUNK, D // 2), jnp.int32),
            pltpu.VMEM((CHUNK, D // 2), jnp.int32),
            pltpu.VMEM((CHUNK, D // 2), jnp.int32),
            pltpu.VMEM((CHUNK, D // 2), jnp.int32),
            pltpu.VMEM((CHUNK, D // 2), jnp.int32),
            pltpu.VMEM((CHUNK, D // 2), jnp.int32),
            pltpu.VMEM((CHUNK, D // 2), jnp.int32),
            pltpu.VMEM((CHUNK, D // 2), jnp.int32),
            pltpu.VMEM((CHUNK, D // 2), jnp.int32),
            pltpu.VMEM((CHUNK, D // 2), jnp.int32),
            pltpu.VMEM((NLANE,), jnp.float32),
            pltpu.SemaphoreType.DMA,
            pltpu.SemaphoreType.DMA,
            pltpu.SemaphoreType.DMA,
            pltpu.SemaphoreType.DMA,
        ],
    )

    h1i, h2i, h3i = _pack_tables(h_c1, h_c2, h_c3)
    partials = run(h1i, h2i, h3i, trt)
    total = jnp.sum(partials) / NLANE
    return total / T + 1e-16


# speedup vs baseline: 1.0172x; 1.0172x over previous
import jax
import jax.numpy as jnp
from jax import lax
from jax.experimental import pallas as pl
from jax.experimental.pallas import tpu as pltpu
from jax.experimental.pallas import tpu_sc as plsc

N = 10000
D = 256
T = 160000
MARGIN = 1.0

NC = 2
NS = 16
NW = NC * NS
TPW = T // NW
CHUNK = 40
NCHUNK = TPW // CHUNK
NBUF = 4
PRE = NBUF - 1
NMAIN = (NCHUNK - PRE) // NBUF
NTAIL = NCHUNK - NBUF * NMAIN
NLANE = 16
NSEG = D // (2 * NLANE)


def _sc_kernel(h1, h2, h3, trt, out,
               idx_v,
               av0, pv0, nv0, av1, pv1, nv1, av2, pv2, nv2, av3, pv3, nv3,
               acc_v, sem0, sem1, sem2, sem3):
    cid = lax.axis_index("c")
    sid = lax.axis_index("s")
    wid = sid * NC + cid
    base = wid * TPW

    pltpu.sync_copy(trt.at[pl.ds(base, TPW)], idx_v.at[pl.ds(0, TPW)])
    pltpu.sync_copy(trt.at[pl.ds(T + base, TPW)], idx_v.at[pl.ds(TPW, TPW)])
    pltpu.sync_copy(trt.at[pl.ds(2 * T + base, TPW)], idx_v.at[pl.ds(2 * TPW, TPW)])

    rings = (
        ((av0, pv0, nv0), sem0),
        ((av1, pv1, nv1), sem1),
        ((av2, pv2, nv2), sem2),
        ((av3, pv3, nv3), sem3),
    )

    def copies(g, bufs, sem):
        off = g * CHUNK
        ba, bp, bn = bufs
        return (
            pltpu.make_async_copy(h1.at[idx_v.at[pl.ds(off, CHUNK)]], ba, sem),
            pltpu.make_async_copy(h2.at[idx_v.at[pl.ds(TPW + off, CHUNK)]], bp, sem),
            pltpu.make_async_copy(h3.at[idx_v.at[pl.ds(2 * TPW + off, CHUNK)]], bn, sem),
        )

    def start(g, bufs, sem):
        for cp in copies(g, bufs, sem):
            cp.start()

    def wait(g, bufs, sem):
        for cp in copies(g, bufs, sem):
            cp.wait()

    def compute(bufs, acc):
        ba, bp, bn = bufs

        mask_hi = jnp.full((NLANE,), -65536, jnp.int32)
        shift16 = jnp.full((NLANE,), 16, jnp.int32)

        def one_triplet(t):
            part = jnp.zeros((NLANE,), jnp.float32)
            for j in range(NSEG):
                sl = pl.ds(j * NLANE, NLANE)
                a = plsc.bitcast(ba[t, sl], jnp.bfloat16)
                p = plsc.bitcast(bp[t, sl], jnp.bfloat16)
                q = plsc.bitcast(bn[t, sl], jnp.bfloat16)
                d1 = a - p
                d2 = a - q
                s = d1 - d2
                w = d1 + d2
                si = plsc.bitcast(s, jnp.int32)
                s_lo = plsc.bitcast(lax.shift_left(si, shift16), jnp.float32)
                s_hi = plsc.bitcast(lax.bitwise_and(si, mask_hi), jnp.float32)
                w_lo, w_hi = plsc.unpack(w, format=plsc.PackFormat.INTERLEAVED)
                part = part + s_lo * w_lo
                part = part + s_hi * w_hi
            c = jnp.sum(part)
            return jnp.maximum(c + MARGIN, 0.0)

        def t_body(t, acc2):
            return acc2 + one_triplet(t)

        return lax.fori_loop(0, CHUNK, t_body, acc)

    for b in range(PRE):
        start(b, *rings[b])

    def ring_body(i, acc):
        g0 = NBUF * i
        for b in range(NBUF):
            g = g0 + b
            bufs, sem = rings[b]
            wait(g, bufs, sem)
            start(g + PRE, *rings[(b + PRE) % NBUF])
            acc = compute(bufs, acc)
        return acc

    acc = lax.fori_loop(0, NMAIN, ring_body, jnp.float32(0.0))
    for k in range(NTAIL):
        g = NBUF * NMAIN + k
        bufs, sem = rings[g % NBUF]
        wait(g, bufs, sem)
        if g + PRE < NCHUNK:
            start(g + PRE, *rings[(g + PRE) % NBUF])
        acc = compute(bufs, acc)
    total = acc

    acc_v[...] = jnp.full((NLANE,), total, jnp.float32)
    pltpu.sync_copy(acc_v, out.at[wid])


@jax.jit
def kernel(h_c1, h_c2, h_c3, triplets):
    trt = triplets.astype(jnp.int32).T.reshape(3 * T)

    mesh = plsc.VectorSubcoreMesh(core_axis_name="c", subcore_axis_name="s",
                                  num_cores=NC, num_subcores=NS)
    run = pl.kernel(
        _sc_kernel,
        out_type=jax.ShapeDtypeStruct((NW, NLANE), jnp.float32),
        mesh=mesh,
        compiler_params=pltpu.CompilerParams(needs_layout_passes=False),
        scratch_types=[
            pltpu.VMEM((3 * TPW,), jnp.int32),
            pltpu.VMEM((CHUNK, D // 2), jnp.int32),
            pltpu.VMEM((CHUNK, D // 2), jnp.int32),
            pltpu.VMEM((CHUNK, D // 2), jnp.int32),
            pltpu.VMEM((CHUNK, D // 2), jnp.int32),
            pltpu.VMEM((CHUNK, D // 2), jnp.int32),
            pltpu.VMEM((CHUNK, D // 2), jnp.int32),
            pltpu.VMEM((CHUNK, D // 2), jnp.int32),
            pltpu.VMEM((CHUNK, D // 2), jnp.int32),
            pltpu.VMEM((CHUNK, D // 2), jnp.int32),
            pltpu.VMEM((CHUNK, D // 2), jnp.int32),
            pltpu.VMEM((CHUNK, D // 2), jnp.int32),
            pltpu.VMEM((CHUNK, D // 2), jnp.int32),
            pltpu.VMEM((NLANE,), jnp.float32),
            pltpu.SemaphoreType.DMA,
            pltpu.SemaphoreType.DMA,
            pltpu.SemaphoreType.DMA,
            pltpu.SemaphoreType.DMA,
        ],
    )

    def to_i32(h):
        u = lax.bitcast_convert_type(h.astype(jnp.bfloat16), jnp.uint16)
        lo = u[:, : D // 2].astype(jnp.uint32)
        hi = u[:, D // 2:].astype(jnp.uint32)
        return lax.bitcast_convert_type(lo | (hi << 16), jnp.int32)

    partials = run(to_i32(h_c1), to_i32(h_c2), to_i32(h_c3), trt)
    total = jnp.sum(partials) / NLANE
    return total / T + 1e-16
